# Initial kernel scaffold; baseline (speedup 1.0000x reference)
import jax
import jax.numpy as jnp
from jax.experimental import pallas as pl


def kernel(src_features, fp_idx, edge_index, edge_w):
    raise NotImplementedError("write your pallas kernel here")


# speedup vs baseline: 49.9883x; 49.9883x over previous
import jax
import jax.numpy as jnp
from jax import lax
from jax.experimental import pallas as pl
from jax.experimental.pallas import tpu as pltpu
from jax.experimental.pallas import tpu_sc as plsc

N = 50000
NP = 50048
C = 2
F = 32
E = 800000
LW = 128
G = 4
NCHUNK = 1564
ERP = NCHUNK * G
EPAD = ERP * LW - E
NT = 16
RPT_Z = NP // NT
RB = 2176
NB = NP // RB
CPT = NCHUNK // NT
CEXTRA = NCHUNK - NT * CPT


def _prep_body(nodes_ref, ew_ref, f0_ref, w0_ref, f1_ref, w1_ref):
    nodes = nodes_ref[...]
    w = 1.0 / jnp.square(ew_ref[...] + 1e-10)
    for cc, fref, wref in ((0, f0_ref, w0_ref), (1, f1_ref, w1_ref)):
        f = nodes[:, cc * F:(cc + 1) * F]
        m = jnp.any(f != 0, axis=1, keepdims=True)
        wm = jnp.where(m, w, 0.0)
        fref[...] = f * wm
        wref[...] = wm


def _fin_body(af0_ref, aw0_ref, af1_ref, aw1_ref, nf_ref, fp_ref, o_ref):
    fpb = fp_ref[...] > 0.5
    nf = nf_ref[...]
    outs = []
    for cc, afr, awr in ((0, af0_ref, aw0_ref), (1, af1_ref, aw1_ref)):
        interp = afr[...] / jnp.maximum(awr[...], 1e-10)
        outs.append(jnp.where(fpb, nf[:, cc * F:(cc + 1) * F], interp))
    o_ref[...] = jnp.concatenate(outs, axis=1)


def _sc_body(vf0, vw0, vf1, vw1, srcs2, dsts2, z2, z1,
             af0, aw0, af1, aw1,
             accf, accw, sbuf, dbuf, frows, wrows, sem_g, sem_s):
    c = lax.axis_index("c")
    s = lax.axis_index("s")

    pltpu.sync_copy(z2, accf.at[pl.ds(s * RPT_Z, RPT_Z)])
    pltpu.sync_copy(z1, accw.at[pl.ds(s * RPT_Z, RPT_Z)])
    plsc.subcore_barrier()

    def phase_b(vf, vw):
        c0 = s * CPT + jnp.minimum(s, CEXTRA)
        cnt = CPT + (s < CEXTRA).astype(jnp.int32)

        def chunk(i, carry):
            base = (c0 + i) * G
            pltpu.sync_copy(srcs2.at[pl.ds(base, G)], sbuf)
            pltpu.sync_copy(dsts2.at[pl.ds(base, G)], dbuf)
            hs = [pltpu.async_copy(vf.at[sbuf.at[j]], frows.at[j], sem_g)
                  for j in range(G)]
            hs += [pltpu.async_copy(vw.at[sbuf.at[j]], wrows.at[j], sem_g)
                   for j in range(G)]
            for h in hs:
                h.wait()
            hs2 = [pltpu.async_copy(frows.at[j], accf.at[dbuf.at[j]], sem_s,
                                    add=True)
                   for j in range(G)]
            hs2 += [pltpu.async_copy(wrows.at[j], accw.at[dbuf.at[j]], sem_s,
                                     add=True)
                    for j in range(G)]
            for h in hs2:
                h.wait()
            return carry

        lax.fori_loop(0, cnt, chunk, 0)

    pl.when(c == 0)(lambda: phase_b(vf0, vw0))
    pl.when(c == 1)(lambda: phase_b(vf1, vw1))
    plsc.subcore_barrier()

    def copyout(outf, outw):
        pltpu.sync_copy(accf.at[pl.ds(s * RPT_Z, RPT_Z)],
                        outf.at[pl.ds(s * RPT_Z, RPT_Z)])
        pltpu.sync_copy(accw.at[pl.ds(s * RPT_Z, RPT_Z)],
                        outw.at[pl.ds(s * RPT_Z, RPT_Z)])

    pl.when(c == 0)(lambda: copyout(af0, aw0))
    pl.when(c == 1)(lambda: copyout(af1, aw1))


@jax.jit
def kernel(src_features, fp_idx, edge_index, edge_w):
    nodes = jnp.zeros((N, C, F), src_features.dtype).at[fp_idx].set(src_features)
    nodes_flat = jnp.pad(nodes.reshape(N, C * F), ((0, NP - N), (0, 0)))
    fpflag = jnp.pad(jnp.zeros((N, 1), jnp.float32).at[fp_idx].set(1.0),
                     ((0, NP - N), (0, 0)))
    ew = jnp.pad(edge_w[:N], ((0, NP - N), (0, 0)))
    srcs2 = jnp.concatenate(
        [edge_index[0], jnp.full((EPAD,), N, jnp.int32)]).reshape(ERP, LW)
    dsts2 = jnp.concatenate(
        [edge_index[1], jnp.full((EPAD,), N, jnp.int32)]).reshape(ERP, LW)
    z2 = jnp.zeros((RPT_Z, F), jnp.float32)
    z1 = jnp.zeros((RPT_Z,), jnp.float32)

    vf0, vw0, vf1, vw1 = pl.pallas_call(
        _prep_body,
        grid=(NB,),
        in_specs=[pl.BlockSpec((RB, C * F), lambda i: (i, 0)),
                  pl.BlockSpec((RB, 1), lambda i: (i, 0))],
        out_specs=[pl.BlockSpec((RB, F), lambda i: (i, 0)),
                   pl.BlockSpec((RB, 1), lambda i: (i, 0))] * 2,
        out_shape=[jax.ShapeDtypeStruct((NP, F), jnp.float32),
                   jax.ShapeDtypeStruct((NP, 1), jnp.float32)] * 2,
    )(nodes_flat, ew)
    vw0_1 = vw0.reshape(NP)
    vw1_1 = vw1.reshape(NP)

    mesh = plsc.VectorSubcoreMesh(core_axis_name="c", subcore_axis_name="s",
                                  num_cores=2, num_subcores=NT)
    af0, aw0, af1, aw1 = pl.kernel(
        _sc_body,
        out_type=[jax.ShapeDtypeStruct((NP, F), jnp.float32),
                  jax.ShapeDtypeStruct((NP,), jnp.float32)] * 2,
        mesh=mesh,
        compiler_params=pltpu.CompilerParams(use_tc_tiling_on_sc=False),
        scratch_types=[
            pltpu.VMEM_SHARED((NP, F), jnp.float32),
            pltpu.VMEM_SHARED((NP,), jnp.float32),
            pltpu.VMEM((G, LW), jnp.int32),
            pltpu.VMEM((G, LW), jnp.int32),
            pltpu.VMEM((G, LW, F), jnp.float32),
            pltpu.VMEM((G, LW), jnp.float32),
            pltpu.SemaphoreType.DMA,
            pltpu.SemaphoreType.DMA,
        ],
    )(vf0, vw0_1, vf1, vw1_1, srcs2, dsts2, z2, z1)

    outflat = pl.pallas_call(
        _fin_body,
        grid=(NB,),
        in_specs=[pl.BlockSpec((RB, F), lambda i: (i, 0)),
                  pl.BlockSpec((RB, 1), lambda i: (i, 0)),
                  pl.BlockSpec((RB, F), lambda i: (i, 0)),
                  pl.BlockSpec((RB, 1), lambda i: (i, 0)),
                  pl.BlockSpec((RB, C * F), lambda i: (i, 0)),
                  pl.BlockSpec((RB, 1), lambda i: (i, 0))],
        out_specs=pl.BlockSpec((RB, C * F), lambda i: (i, 0)),
        out_shape=jax.ShapeDtypeStruct((NP, C * F), jnp.float32),
    )(af0, aw0.reshape(NP, 1), af1, aw1.reshape(NP, 1), nodes_flat, fpflag)
    return outflat[:N].reshape(N, C, F)


# speedup vs baseline: 60.4311x; 1.2089x over previous
import jax
import jax.numpy as jnp
from jax import lax
from jax.experimental import pallas as pl
from jax.experimental.pallas import tpu as pltpu
from jax.experimental.pallas import tpu_sc as plsc

N = 50000
NP = 50048
C = 2
F = 32
E = 800000
NS = 12500
NSP = 12544
SR = NSP // 128
RA_PT = SR // 16
RA_EX = SR - 16 * RA_PT
LW = 128
G = 4
NCHUNK = 1564
ERP = NCHUNK * G
EPAD = ERP * LW - E
NT = 16
RPT_Z = NP // NT
RB = 2176
NB = NP // RB
CPT = NCHUNK // NT
CEXTRA = NCHUNK - NT * CPT


def _fin_body(af0_ref, aw0_ref, af1_ref, aw1_ref, n0_ref, n1_ref, fp_ref,
              o_ref):
    fpb = fp_ref[...] > 0.5
    outs = []
    for afr, awr, nfr in ((af0_ref, aw0_ref, n0_ref),
                          (af1_ref, aw1_ref, n1_ref)):
        interp = afr[...] / jnp.maximum(awr[...], 1e-10)
        outs.append(jnp.where(fpb, nfr[...], interp))
    o_ref[...] = jnp.concatenate(outs, axis=1)


def _sc_prep_body(sf0, sf1, tgt2, perm2, mk0, mk1, ewN, z2, z1,
                  vf0, vw0, vf1, vw1, n0, n1, flag,
                  tbuf, pbuf, rowbuf, ewrow, mrow, wmbuf, onesb,
                  sem_g, sem_s):
    c = lax.axis_index("c")
    s = lax.axis_index("s")

    def zfill0():
        pltpu.sync_copy(z2, n0.at[pl.ds(s * RPT_Z, RPT_Z)])
        pltpu.sync_copy(z2, vf0.at[pl.ds(s * RPT_Z, RPT_Z)])
        pltpu.sync_copy(z1, vw0.at[pl.ds(s * RPT_Z, RPT_Z)])
        pltpu.sync_copy(z1, flag.at[pl.ds(s * RPT_Z, RPT_Z)])

    def zfill1():
        pltpu.sync_copy(z2, n1.at[pl.ds(s * RPT_Z, RPT_Z)])
        pltpu.sync_copy(z2, vf1.at[pl.ds(s * RPT_Z, RPT_Z)])
        pltpu.sync_copy(z1, vw1.at[pl.ds(s * RPT_Z, RPT_Z)])

    pl.when(c == 0)(zfill0)
    pl.when(c == 1)(zfill1)
    for j in range(8):
        onesb[pl.ds(16 * j, 16)] = jnp.full((16,), 1.0, jnp.float32)
    plsc.subcore_barrier()

    def scatter_phase(sfc, mkc, nc, vfc, vwc, do_flag):
        r0 = s * RA_PT + jnp.minimum(s, RA_EX)
        cnt = RA_PT + (s < RA_EX).astype(jnp.int32)
        rvecs = [g * 16 + lax.iota(jnp.int32, 16) for g in range(8)]

        def row(r, carry):
            pltpu.sync_copy(tgt2.at[pl.ds(r, 1)], tbuf)
            pltpu.sync_copy(perm2.at[pl.ds(r, 1)], pbuf)
            pltpu.sync_copy(mkc.at[pl.ds(r, 1)], mrow)
            pltpu.async_copy(sfc.at[pbuf.at[0]], rowbuf, sem_g).wait()
            pltpu.async_copy(ewN.at[tbuf.at[0]], ewrow, sem_g).wait()
            pltpu.async_copy(rowbuf, nc.at[tbuf.at[0]], sem_s).wait()
            if do_flag:
                pltpu.async_copy(onesb, flag.at[tbuf.at[0]], sem_s).wait()
            for g in range(8):
                ev = ewrow[pl.ds(g * 16, 16)]
                mv = mrow[0, pl.ds(g * 16, 16)]
                evs = ev + 1e-10
                wm = mv / (evs * evs)
                wmbuf[pl.ds(g * 16, 16)] = wm
                for k in range(F):
                    kv = jnp.full((16,), k, jnp.int32)
                    gv = plsc.load_gather(rowbuf, [rvecs[g], kv])
                    plsc.store_scatter(rowbuf, [rvecs[g], kv], gv * wm)
            pltpu.async_copy(rowbuf, vfc.at[tbuf.at[0]], sem_s).wait()
            pltpu.async_copy(wmbuf, vwc.at[tbuf.at[0]], sem_s).wait()
            return carry

        lax.fori_loop(r0, r0 + cnt, row, 0)

    pl.when(c == 0)(lambda: scatter_phase(sf0, mk0, n0, vf0, vw0, True))
    pl.when(c == 1)(lambda: scatter_phase(sf1, mk1, n1, vf1, vw1, False))


def _sc_body(vf0, vw0, vf1, vw1, srcs2, dsts2, z2, z1,
             af0, aw0, af1, aw1,
             accf, accw, sbuf, dbuf, frows, wrows, sem_g, sem_s):
    c = lax.axis_index("c")
    s = lax.axis_index("s")

    pltpu.sync_copy(z2, accf.at[pl.ds(s * RPT_Z, RPT_Z)])
    pltpu.sync_copy(z1, accw.at[pl.ds(s * RPT_Z, RPT_Z)])
    plsc.subcore_barrier()

    def phase_b(vf, vw):
        c0 = s * CPT + jnp.minimum(s, CEXTRA)
        cnt = CPT + (s < CEXTRA).astype(jnp.int32)

        def chunk(i, carry):
            base = (c0 + i) * G
            pltpu.sync_copy(srcs2.at[pl.ds(base, G)], sbuf)
            pltpu.sync_copy(dsts2.at[pl.ds(base, G)], dbuf)
            hs = [pltpu.async_copy(vf.at[sbuf.at[j]], frows.at[j], sem_g)
                  for j in range(G)]
            hs += [pltpu.async_copy(vw.at[sbuf.at[j]], wrows.at[j], sem_g)
                   for j in range(G)]
            for h in hs:
                h.wait()
            hs2 = [pltpu.async_copy(frows.at[j], accf.at[dbuf.at[j]], sem_s,
                                    add=True)
                   for j in range(G)]
            hs2 += [pltpu.async_copy(wrows.at[j], accw.at[dbuf.at[j]], sem_s,
                                     add=True)
                    for j in range(G)]
            for h in hs2:
                h.wait()
            return carry

        lax.fori_loop(0, cnt, chunk, 0)

    pl.when(c == 0)(lambda: phase_b(vf0, vw0))
    pl.when(c == 1)(lambda: phase_b(vf1, vw1))
    plsc.subcore_barrier()

    def copyout(outf, outw):
        pltpu.sync_copy(accf.at[pl.ds(s * RPT_Z, RPT_Z)],
                        outf.at[pl.ds(s * RPT_Z, RPT_Z)])
        pltpu.sync_copy(accw.at[pl.ds(s * RPT_Z, RPT_Z)],
                        outw.at[pl.ds(s * RPT_Z, RPT_Z)])

    pl.when(c == 0)(lambda: copyout(af0, aw0))
    pl.when(c == 1)(lambda: copyout(af1, aw1))


@jax.jit
def kernel(src_features, fp_idx, edge_index, edge_w):
    m0 = jnp.any(src_features[:, 0, :] != 0, axis=1).astype(jnp.float32)
    m1 = jnp.any(src_features[:, 1, :] != 0, axis=1).astype(jnp.float32)
    iota = jnp.arange(NS, dtype=jnp.int32)
    sfp, perm, mk0, mk1 = lax.sort((fp_idx, iota, m0, m1), num_keys=1,
                                   is_stable=True)
    iswin = jnp.concatenate([sfp[:-1] != sfp[1:],
                             jnp.ones((1,), bool)])
    tgt = jnp.where(iswin, sfp, N)
    tgt2 = jnp.concatenate(
        [tgt, jnp.full((NSP - NS,), N, jnp.int32)]).reshape(SR, LW)
    perm2 = jnp.concatenate(
        [perm, jnp.zeros((NSP - NS,), jnp.int32)]).reshape(SR, LW)
    mk02 = jnp.concatenate([mk0, jnp.zeros((NSP - NS,))]).reshape(SR, LW)
    mk12 = jnp.concatenate([mk1, jnp.zeros((NSP - NS,))]).reshape(SR, LW)
    sf0 = jnp.pad(src_features[:, 0, :], ((0, NSP - NS), (0, 0)))
    sf1 = jnp.pad(src_features[:, 1, :], ((0, NSP - NS), (0, 0)))
    ewN = jnp.pad(edge_w[:N, 0], (0, NP - N))
    srcs2 = jnp.concatenate(
        [edge_index[0], jnp.full((EPAD,), N, jnp.int32)]).reshape(ERP, LW)
    dsts2 = jnp.concatenate(
        [edge_index[1], jnp.full((EPAD,), N, jnp.int32)]).reshape(ERP, LW)
    z2 = jnp.zeros((RPT_Z, F), jnp.float32)
    z1 = jnp.zeros((RPT_Z,), jnp.float32)

    mesh = plsc.VectorSubcoreMesh(core_axis_name="c", subcore_axis_name="s",
                                  num_cores=2, num_subcores=NT)

    vf0, vw0, vf1, vw1, n0, n1, flag = pl.kernel(
        _sc_prep_body,
        out_type=[jax.ShapeDtypeStruct((NP, F), jnp.float32),
                  jax.ShapeDtypeStruct((NP,), jnp.float32)] * 2 +
                 [jax.ShapeDtypeStruct((NP, F), jnp.float32)] * 2 +
                 [jax.ShapeDtypeStruct((NP,), jnp.float32)],
        mesh=mesh,
        compiler_params=pltpu.CompilerParams(use_tc_tiling_on_sc=False,
                                             needs_layout_passes=False),
        scratch_types=[
            pltpu.VMEM((1, LW), jnp.int32),
            pltpu.VMEM((1, LW), jnp.int32),
            pltpu.VMEM((LW, F), jnp.float32),
            pltpu.VMEM((LW,), jnp.float32),
            pltpu.VMEM((1, LW), jnp.float32),
            pltpu.VMEM((LW,), jnp.float32),
            pltpu.VMEM((LW,), jnp.float32),
            pltpu.SemaphoreType.DMA,
            pltpu.SemaphoreType.DMA,
        ],
    )(sf0, sf1, tgt2, perm2, mk02, mk12, ewN, z2, z1)

    af0, aw0, af1, aw1 = pl.kernel(
        _sc_body,
        out_type=[jax.ShapeDtypeStruct((NP, F), jnp.float32),
                  jax.ShapeDtypeStruct((NP,), jnp.float32)] * 2,
        mesh=mesh,
        compiler_params=pltpu.CompilerParams(use_tc_tiling_on_sc=False),
        scratch_types=[
            pltpu.VMEM_SHARED((NP, F), jnp.float32),
            pltpu.VMEM_SHARED((NP,), jnp.float32),
            pltpu.VMEM((G, LW), jnp.int32),
            pltpu.VMEM((G, LW), jnp.int32),
            pltpu.VMEM((G, LW, F), jnp.float32),
            pltpu.VMEM((G, LW), jnp.float32),
            pltpu.SemaphoreType.DMA,
            pltpu.SemaphoreType.DMA,
        ],
    )(vf0, vw0, vf1, vw1, srcs2, dsts2, z2, z1)

    outflat = pl.pallas_call(
        _fin_body,
        grid=(NB,),
        in_specs=[pl.BlockSpec((RB, F), lambda i: (i, 0)),
                  pl.BlockSpec((RB, 1), lambda i: (i, 0)),
                  pl.BlockSpec((RB, F), lambda i: (i, 0)),
                  pl.BlockSpec((RB, 1), lambda i: (i, 0)),
                  pl.BlockSpec((RB, F), lambda i: (i, 0)),
                  pl.BlockSpec((RB, F), lambda i: (i, 0)),
                  pl.BlockSpec((RB, 1), lambda i: (i, 0))],
        out_specs=pl.BlockSpec((RB, C * F), lambda i: (i, 0)),
        out_shape=jax.ShapeDtypeStruct((NP, C * F), jnp.float32),
    )(af0, aw0.reshape(NP, 1), af1, aw1.reshape(NP, 1), n0, n1,
      flag.reshape(NP, 1))
    return outflat[:N].reshape(N, C, F)
